# Initial kernel scaffold; baseline (speedup 1.0000x reference)
import jax
import jax.numpy as jnp
from jax.experimental import pallas as pl


def kernel(x, emb_matrix):
    raise NotImplementedError("write your pallas kernel here")


# speedup vs baseline: 1.4596x; 1.4596x over previous
import functools

import jax
import jax.numpy as jnp
from jax import lax
from jax.experimental import pallas as pl
from jax.experimental.pallas import tpu as pltpu
from jax.experimental.pallas import tpu_sc as plsc

_EMB = 32
_NC = 2
_NS = 16
_NW = _NC * _NS
_IDX_MINOR = 128
_K_PER = 8
_CHUNK = _K_PER * _IDX_MINOR


@functools.lru_cache(maxsize=None)
def _make_gather(n_tokens: int):
    b_per_w = n_tokens // _NW
    n_chunks = b_per_w // _CHUNK
    assert b_per_w * _NW == n_tokens and n_chunks * _CHUNK == b_per_w
    idx_rows_per_w = b_per_w // _IDX_MINOR
    mesh = plsc.VectorSubcoreMesh(core_axis_name="c", subcore_axis_name="s")

    @functools.partial(
        pl.kernel,
        out_type=jax.ShapeDtypeStruct((n_tokens, _EMB), jnp.float32),
        mesh=mesh,
        compiler_params=pltpu.CompilerParams(use_tc_tiling_on_sc=False),
        scratch_types=[
            pltpu.VMEM((_K_PER, _IDX_MINOR), jnp.int32),
            pltpu.VMEM((_CHUNK, _EMB), jnp.float32),
            pltpu.SemaphoreType.DMA,
        ],
    )
    def body(idx_hbm, table_hbm, out_hbm, idx_v, rows_v, sem):
        wid = lax.axis_index("s") * _NC + lax.axis_index("c")

        def chunk_body(g, carry):
            base = wid * b_per_w + g * _CHUNK
            idx_row0 = wid * idx_rows_per_w + g * _K_PER
            pltpu.sync_copy(idx_hbm.at[pl.ds(idx_row0, _K_PER)], idx_v)
            copies = [
                pltpu.async_copy(
                    table_hbm.at[idx_v.at[j]],
                    rows_v.at[pl.ds(j * _IDX_MINOR, _IDX_MINOR)],
                    sem,
                )
                for j in range(_K_PER)
            ]
            for c in copies:
                c.wait()
            pltpu.sync_copy(rows_v, out_hbm.at[pl.ds(base, _CHUNK)])
            return carry

        lax.fori_loop(0, n_chunks, chunk_body, 0)

    return body


def kernel(x, emb_matrix):
    b, l = x.shape
    n = b * l
    idx2d = x.reshape(n // _IDX_MINOR, _IDX_MINOR).astype(jnp.int32)
    out = _make_gather(n)(idx2d, emb_matrix)
    return out.reshape(b, l, _EMB)


# speedup vs baseline: 1.4895x; 1.0205x over previous
import functools

import jax
import jax.numpy as jnp
from jax import lax
from jax.experimental import pallas as pl
from jax.experimental.pallas import tpu as pltpu
from jax.experimental.pallas import tpu_sc as plsc

_EMB = 32
_NC = 2
_NS = 16
_NW = _NC * _NS
_IDX_MINOR = 128
_K_PER = 10
_CHUNK = _K_PER * _IDX_MINOR


@functools.lru_cache(maxsize=None)
def _make_gather(n_tokens: int):
    b_per_w = n_tokens // _NW
    n_chunks = b_per_w // _CHUNK
    assert b_per_w * _NW == n_tokens and n_chunks * _CHUNK == b_per_w
    assert n_chunks % 2 == 0
    idx_rows_per_w = b_per_w // _IDX_MINOR
    mesh = plsc.VectorSubcoreMesh(core_axis_name="c", subcore_axis_name="s")

    @functools.partial(
        pl.kernel,
        out_type=jax.ShapeDtypeStruct((n_tokens, _EMB), jnp.float32),
        mesh=mesh,
        compiler_params=pltpu.CompilerParams(use_tc_tiling_on_sc=False),
        scratch_types=[
            pltpu.VMEM((2, _K_PER, _IDX_MINOR), jnp.int32),
            pltpu.VMEM((2, _CHUNK, _EMB), jnp.float32),
            pltpu.SemaphoreType.DMA,
            pltpu.SemaphoreType.DMA,
            pltpu.SemaphoreType.DMA,
            pltpu.SemaphoreType.DMA,
        ],
    )
    def body(idx_hbm, table_hbm, out_hbm, idx_v, rows_v, g0, g1, o0, o1):
        wid = lax.axis_index("s") * _NC + lax.axis_index("c")
        gsems = (g0, g1)
        osems = (o0, o1)

        def fire_gather(g, par, sem):
            idx_row0 = wid * idx_rows_per_w + g * _K_PER
            pltpu.sync_copy(idx_hbm.at[pl.ds(idx_row0, _K_PER)], idx_v.at[par])
            for j in range(_K_PER):
                pltpu.async_copy(
                    table_hbm.at[idx_v.at[par, j]],
                    rows_v.at[par, pl.ds(j * _IDX_MINOR, _IDX_MINOR)],
                    sem,
                )

        def drain_gather(par, sem):
            for j in range(_K_PER):
                pltpu.make_async_copy(
                    table_hbm.at[idx_v.at[par, j]],
                    rows_v.at[par, pl.ds(j * _IDX_MINOR, _IDX_MINOR)],
                    sem,
                ).wait()

        def wb_copy(g, par, sem):
            base = wid * b_per_w + g * _CHUNK
            return pltpu.make_async_copy(
                rows_v.at[par], out_hbm.at[pl.ds(base, _CHUNK)], sem
            )

        fire_gather(0, 0, gsems[0])
        fire_gather(1, 1, gsems[1])

        def loop_body(h, carry):
            for par in range(2):
                g = 2 * h + par
                drain_gather(par, gsems[par])
                wb_copy(g, par, osems[par]).start()

                @pl.when(g + 2 < n_chunks)
                def _(g=g, par=par):
                    wb_copy(g, par, osems[par]).wait()
                    fire_gather(g + 2, par, gsems[par])

            return carry

        lax.fori_loop(0, n_chunks // 2, loop_body, 0)
        wb_copy(n_chunks - 2, 0, osems[0]).wait()
        wb_copy(n_chunks - 1, 1, osems[1]).wait()

    return body


def kernel(x, emb_matrix):
    b, l = x.shape
    n = b * l
    idx2d = x.reshape(n // _IDX_MINOR, _IDX_MINOR).astype(jnp.int32)
    out = _make_gather(n)(idx2d, emb_matrix)
    return out.reshape(b, l, _EMB)
